# baseline (speedup 1.0000x reference)
import functools

import jax
import jax.numpy as jnp
from jax.experimental import pallas as pl


def _conv_stats_body(C, S, W, W2, xf_ref, wd_ref, wk_ref, hd_ref, hk_ref,
                     st_ref):
    i = pl.program_id(0)
    accd = jnp.zeros((C, S), jnp.float32)
    acck = jnp.zeros((C, S), jnp.float32)
    for dy in range(3):
        for dx in range(3):
            k = dy * 3 + dx
            off = dy * W2 + dx
            s = xf_ref[0, :, off:off + S]
            accd = accd + jnp.dot(wd_ref[k], s,
                                  preferred_element_type=jnp.float32)
            acck = acck + jnp.dot(wk_ref[k], s,
                                  preferred_element_type=jnp.float32)
    hd_ref[0] = accd
    hk_ref[0] = acck

    col = jax.lax.broadcasted_iota(jnp.int32, (C, S), 1)
    valid = (col % W2) < W
    md = jnp.where(valid, accd, 0.0)
    mk = jnp.where(valid, acck, 0.0)
    st = jnp.concatenate([
        jnp.sum(md, axis=1, keepdims=True),
        jnp.sum(md * md, axis=1, keepdims=True),
        jnp.sum(mk, axis=1, keepdims=True),
        jnp.sum(mk * mk, axis=1, keepdims=True),
    ], axis=1)

    @pl.when(i == 0)
    def _():
        st_ref[...] = jnp.zeros_like(st_ref)

    st_ref[...] += st


def _bn_silu_proj_body(nv, eps, hd_ref, hk_ref, st_ref, gb_ref, wd_ref,
                       wk_ref, bd_ref, bk_ref, od_ref, ok_ref):
    st = st_ref[...]
    gb = gb_ref[...]

    mean_d = st[:, 0:1] / nv
    var_d = st[:, 1:2] / nv - mean_d * mean_d
    scale_d = gb[:, 0:1] * jax.lax.rsqrt(var_d + eps)
    shift_d = gb[:, 1:2] - mean_d * scale_d
    yd = hd_ref[0] * scale_d + shift_d
    yd = yd * jax.nn.sigmoid(yd)
    od_ref[0] = jnp.dot(wd_ref[...], yd,
                        preferred_element_type=jnp.float32) + bd_ref[...]

    mean_k = st[:, 2:3] / nv
    var_k = st[:, 3:4] / nv - mean_k * mean_k
    scale_k = gb[:, 2:3] * jax.lax.rsqrt(var_k + eps)
    shift_k = gb[:, 3:4] - mean_k * scale_k
    yk = hk_ref[0] * scale_k + shift_k
    yk = yk * jax.nn.sigmoid(yk)
    ok_ref[0] = jnp.dot(wk_ref[...], yk,
                        preferred_element_type=jnp.float32) + bk_ref[...]


def _head_level(x, pd, pk, interpret=False):
    B, C, H, W = x.shape
    W2 = W + 2
    S = H * W2
    SPAD = (H + 2) * W2 + 2
    CD = pd["w2"].shape[0]
    CK = pk["w2"].shape[0]

    xf = jnp.pad(x, ((0, 0), (0, 0), (1, 1), (1, 1)))
    xf = xf.reshape(B, C, (H + 2) * W2)
    xf = jnp.pad(xf, ((0, 0), (0, 0), (0, 2)))

    wd1 = jnp.transpose(pd["w1"], (2, 3, 0, 1)).reshape(9, C, C)
    wk1 = jnp.transpose(pk["w1"], (2, 3, 0, 1)).reshape(9, C, C)

    hd, hk, st = pl.pallas_call(
        functools.partial(_conv_stats_body, C, S, W, W2),
        grid=(B,),
        in_specs=[
            pl.BlockSpec((1, C, SPAD), lambda i: (i, 0, 0)),
            pl.BlockSpec((9, C, C), lambda i: (0, 0, 0)),
            pl.BlockSpec((9, C, C), lambda i: (0, 0, 0)),
        ],
        out_specs=[
            pl.BlockSpec((1, C, S), lambda i: (i, 0, 0)),
            pl.BlockSpec((1, C, S), lambda i: (i, 0, 0)),
            pl.BlockSpec((C, 4), lambda i: (0, 0)),
        ],
        out_shape=[
            jax.ShapeDtypeStruct((B, C, S), jnp.float32),
            jax.ShapeDtypeStruct((B, C, S), jnp.float32),
            jax.ShapeDtypeStruct((C, 4), jnp.float32),
        ],
        interpret=interpret,
    )(xf, wd1, wk1)

    gb = jnp.stack([pd["gamma"], pd["beta"], pk["gamma"], pk["beta"]],
                   axis=1)
    w2d = pd["w2"].reshape(CD, C)
    w2k = pk["w2"].reshape(CK, C)
    b2d = pd["b2"].reshape(CD, 1)
    b2k = pk["b2"].reshape(CK, 1)

    od, ok = pl.pallas_call(
        functools.partial(_bn_silu_proj_body, float(B * H * W), 1e-5),
        grid=(B,),
        in_specs=[
            pl.BlockSpec((1, C, S), lambda i: (i, 0, 0)),
            pl.BlockSpec((1, C, S), lambda i: (i, 0, 0)),
            pl.BlockSpec((C, 4), lambda i: (0, 0)),
            pl.BlockSpec((C, 4), lambda i: (0, 0)),
            pl.BlockSpec((CD, C), lambda i: (0, 0)),
            pl.BlockSpec((CK, C), lambda i: (0, 0)),
            pl.BlockSpec((CD, 1), lambda i: (0, 0)),
            pl.BlockSpec((CK, 1), lambda i: (0, 0)),
        ],
        out_specs=[
            pl.BlockSpec((1, CD, S), lambda i: (i, 0, 0)),
            pl.BlockSpec((1, CK, S), lambda i: (i, 0, 0)),
        ],
        out_shape=[
            jax.ShapeDtypeStruct((B, CD, S), jnp.float32),
            jax.ShapeDtypeStruct((B, CK, S), jnp.float32),
        ],
        interpret=interpret,
    )(hd, hk, st, gb, w2d, w2k, b2d, b2k)

    det = od.reshape(B, CD, H, W2)[:, :, :, :W]
    kp = ok.reshape(B, CK, H, W2)[:, :, :, :W]
    return det, kp


def kernel(p3, p4, p5, params):
    det3, kp3 = _head_level(p3, params["det3"], params["kp3"])
    det4, kp4 = _head_level(p4, params["det4"], params["kp4"])
    det5, kp5 = _head_level(p5, params["det5"], params["kp5"])
    return (det3, det4, det5, kp3, kp4, kp5)


# speedup vs baseline: 1.0463x; 1.0463x over previous
import functools

import jax
import jax.numpy as jnp
from jax.experimental import pallas as pl
from jax.experimental.pallas import tpu as pltpu


def _conv_stats_body(C, S, H, W, W2, x_ref, wd_ref, wk_ref, hd_ref, hk_ref,
                     st_ref, xs_ref):
    i = pl.program_id(0)
    xs_ref[...] = jnp.zeros_like(xs_ref)
    xs_ref[:, 1:H + 1, 1:W + 1] = x_ref[0]
    xf = xs_ref[...].reshape(C, (H + 3) * W2)
    accd = jnp.zeros((C, S), jnp.float32)
    acck = jnp.zeros((C, S), jnp.float32)
    for dy in range(3):
        for dx in range(3):
            k = dy * 3 + dx
            off = dy * W2 + dx
            s = jax.lax.slice(xf, (0, off), (C, off + S))
            accd = accd + jnp.dot(wd_ref[k], s,
                                  preferred_element_type=jnp.float32)
            acck = acck + jnp.dot(wk_ref[k], s,
                                  preferred_element_type=jnp.float32)
    hd_ref[0] = accd
    hk_ref[0] = acck

    col = jax.lax.broadcasted_iota(jnp.int32, (C, S), 1)
    valid = (col % W2) < W
    md = jnp.where(valid, accd, 0.0)
    mk = jnp.where(valid, acck, 0.0)
    st = jnp.concatenate([
        jnp.sum(md, axis=1, keepdims=True),
        jnp.sum(md * md, axis=1, keepdims=True),
        jnp.sum(mk, axis=1, keepdims=True),
        jnp.sum(mk * mk, axis=1, keepdims=True),
    ], axis=1)

    @pl.when(i == 0)
    def _():
        st_ref[...] = jnp.zeros_like(st_ref)

    st_ref[...] += st


def _bn_silu_proj_body(H, W, W2, nv, eps, hd_ref, hk_ref, st_ref, gb_ref,
                       wd_ref, wk_ref, bd_ref, bk_ref, od_ref, ok_ref):
    st = st_ref[...]
    gb = gb_ref[...]
    CD = od_ref.shape[1]
    CK = ok_ref.shape[1]

    mean_d = st[:, 0:1] / nv
    var_d = st[:, 1:2] / nv - mean_d * mean_d
    scale_d = gb[:, 0:1] * jax.lax.rsqrt(var_d + eps)
    shift_d = gb[:, 1:2] - mean_d * scale_d
    yd = hd_ref[0] * scale_d + shift_d
    yd = yd * jax.nn.sigmoid(yd)
    od = jnp.dot(wd_ref[...], yd,
                 preferred_element_type=jnp.float32) + bd_ref[...]
    od_ref[0] = jax.lax.slice(od.reshape(CD, H, W2), (0, 0, 0), (CD, H, W))

    mean_k = st[:, 2:3] / nv
    var_k = st[:, 3:4] / nv - mean_k * mean_k
    scale_k = gb[:, 2:3] * jax.lax.rsqrt(var_k + eps)
    shift_k = gb[:, 3:4] - mean_k * scale_k
    yk = hk_ref[0] * scale_k + shift_k
    yk = yk * jax.nn.sigmoid(yk)
    ok = jnp.dot(wk_ref[...], yk,
                 preferred_element_type=jnp.float32) + bk_ref[...]
    ok_ref[0] = jax.lax.slice(ok.reshape(CK, H, W2), (0, 0, 0), (CK, H, W))


def _head_level(x, pd, pk, interpret=False):
    B, C, H, W = x.shape
    W2 = W + 2
    S = H * W2
    CD = pd["w2"].shape[0]
    CK = pk["w2"].shape[0]

    wd1 = jnp.transpose(pd["w1"], (2, 3, 0, 1)).reshape(9, C, C)
    wk1 = jnp.transpose(pk["w1"], (2, 3, 0, 1)).reshape(9, C, C)

    hd, hk, st = pl.pallas_call(
        functools.partial(_conv_stats_body, C, S, H, W, W2),
        grid=(B,),
        in_specs=[
            pl.BlockSpec((1, C, H, W), lambda i: (i, 0, 0, 0)),
            pl.BlockSpec((9, C, C), lambda i: (0, 0, 0)),
            pl.BlockSpec((9, C, C), lambda i: (0, 0, 0)),
        ],
        out_specs=[
            pl.BlockSpec((1, C, S), lambda i: (i, 0, 0)),
            pl.BlockSpec((1, C, S), lambda i: (i, 0, 0)),
            pl.BlockSpec((C, 4), lambda i: (0, 0)),
        ],
        out_shape=[
            jax.ShapeDtypeStruct((B, C, S), jnp.float32),
            jax.ShapeDtypeStruct((B, C, S), jnp.float32),
            jax.ShapeDtypeStruct((C, 4), jnp.float32),
        ],
        scratch_shapes=[pltpu.VMEM((C, H + 3, W2), jnp.float32)],
        interpret=interpret,
    )(x, wd1, wk1)

    gb = jnp.stack([pd["gamma"], pd["beta"], pk["gamma"], pk["beta"]],
                   axis=1)
    w2d = pd["w2"].reshape(CD, C)
    w2k = pk["w2"].reshape(CK, C)
    b2d = pd["b2"].reshape(CD, 1)
    b2k = pk["b2"].reshape(CK, 1)

    det, kp = pl.pallas_call(
        functools.partial(_bn_silu_proj_body, H, W, W2, float(B * H * W),
                          1e-5),
        grid=(B,),
        in_specs=[
            pl.BlockSpec((1, C, S), lambda i: (i, 0, 0)),
            pl.BlockSpec((1, C, S), lambda i: (i, 0, 0)),
            pl.BlockSpec((C, 4), lambda i: (0, 0)),
            pl.BlockSpec((C, 4), lambda i: (0, 0)),
            pl.BlockSpec((CD, C), lambda i: (0, 0)),
            pl.BlockSpec((CK, C), lambda i: (0, 0)),
            pl.BlockSpec((CD, 1), lambda i: (0, 0)),
            pl.BlockSpec((CK, 1), lambda i: (0, 0)),
        ],
        out_specs=[
            pl.BlockSpec((1, CD, H, W), lambda i: (i, 0, 0, 0)),
            pl.BlockSpec((1, CK, H, W), lambda i: (i, 0, 0, 0)),
        ],
        out_shape=[
            jax.ShapeDtypeStruct((B, CD, H, W), jnp.float32),
            jax.ShapeDtypeStruct((B, CK, H, W), jnp.float32),
        ],
        interpret=interpret,
    )(hd, hk, st, gb, w2d, w2k, b2d, b2k)
    return det, kp


def kernel(p3, p4, p5, params):
    det3, kp3 = _head_level(p3, params["det3"], params["kp3"])
    det4, kp4 = _head_level(p4, params["det4"], params["kp4"])
    det5, kp5 = _head_level(p5, params["det5"], params["kp5"])
    return (det3, det4, det5, kp3, kp4, kp5)
